# Initial kernel scaffold; baseline (speedup 1.0000x reference)
import jax
import jax.numpy as jnp
from jax.experimental import pallas as pl


def kernel(input_pos, k_val, v_val, k_cache, v_cache):
    raise NotImplementedError("write your pallas kernel here")


# speedup vs baseline: 1.0263x; 1.0263x over previous
import jax
import jax.numpy as jnp
from jax.experimental import pallas as pl
from jax.experimental.pallas import tpu as pltpu

_B, _H, _S, _D = 8, 16, 4096, 128
_Q = 16
_BH = _B * _H


def _body(pos_ref, kc_ref, vc_ref, kv_ref, vv_ref, ko_ref, vo_ref):
    ko_ref[...] = kc_ref[...]
    vo_ref[...] = vc_ref[...]
    row_ids = jax.lax.broadcasted_iota(jnp.int32, (8, _D), 0)
    for q in range(_Q):
        p = pos_ref[q]
        base = (p // 8) * 8
        sel = row_ids == (p - base)
        for val_ref, out_ref in ((kv_ref, ko_ref), (vv_ref, vo_ref)):
            row = jnp.broadcast_to(val_ref[0, pl.ds(q, 1), :], (8, _D))
            chunk = out_ref[0, pl.ds(base, 8), :]
            out_ref[0, pl.ds(base, 8), :] = jnp.where(sel, row, chunk)


def kernel(input_pos, k_val, v_val, k_cache, v_cache):
    kc = k_cache.reshape(_BH, _S, _D)
    vc = v_cache.reshape(_BH, _S, _D)
    kv = k_val.reshape(_BH, _Q, _D)
    vv = v_val.reshape(_BH, _Q, _D)
    grid_spec = pltpu.PrefetchScalarGridSpec(
        num_scalar_prefetch=1,
        grid=(_BH,),
        in_specs=[
            pl.BlockSpec((1, _S, _D), lambda i, pos: (i, 0, 0)),
            pl.BlockSpec((1, _S, _D), lambda i, pos: (i, 0, 0)),
            pl.BlockSpec((1, _Q, _D), lambda i, pos: (i, 0, 0)),
            pl.BlockSpec((1, _Q, _D), lambda i, pos: (i, 0, 0)),
        ],
        out_specs=[
            pl.BlockSpec((1, _S, _D), lambda i, pos: (i, 0, 0)),
            pl.BlockSpec((1, _S, _D), lambda i, pos: (i, 0, 0)),
        ],
    )
    ko, vo = pl.pallas_call(
        _body,
        grid_spec=grid_spec,
        out_shape=[
            jax.ShapeDtypeStruct((_BH, _S, _D), k_cache.dtype),
            jax.ShapeDtypeStruct((_BH, _S, _D), v_cache.dtype),
        ],
    )(input_pos, kc, vc, kv, vv)
    return ko.reshape(_B, _H, _S, _D), vo.reshape(_B, _H, _S, _D)


# speedup vs baseline: 1.1623x; 1.1326x over previous
import jax
import jax.numpy as jnp
from jax.experimental import pallas as pl
from jax.experimental.pallas import tpu as pltpu

_B, _H, _S, _D = 8, 16, 4096, 128
_Q = 16
_BH = _B * _H
_BB = 4


def _body(pos_ref, kc_ref, vc_ref, kv_ref, vv_ref, ko_ref, vo_ref):
    ko_ref[...] = kc_ref[...]
    vo_ref[...] = vc_ref[...]
    row_ids = jax.lax.broadcasted_iota(jnp.int32, (_BB, 8, _D), 1)
    for q in range(_Q):
        p = pos_ref[q]
        base = (p // 8) * 8
        sel = row_ids == (p - base)
        for val_ref, out_ref in ((kv_ref, ko_ref), (vv_ref, vo_ref)):
            row = jnp.broadcast_to(val_ref[:, pl.ds(q, 1), :], (_BB, 8, _D))
            chunk = out_ref[:, pl.ds(base, 8), :]
            out_ref[:, pl.ds(base, 8), :] = jnp.where(sel, row, chunk)


def kernel(input_pos, k_val, v_val, k_cache, v_cache):
    kc = k_cache.reshape(_BH, _S, _D)
    vc = v_cache.reshape(_BH, _S, _D)
    kv = k_val.reshape(_BH, _Q, _D)
    vv = v_val.reshape(_BH, _Q, _D)
    grid_spec = pltpu.PrefetchScalarGridSpec(
        num_scalar_prefetch=1,
        grid=(_BH // _BB,),
        in_specs=[
            pl.BlockSpec((_BB, _S, _D), lambda i, pos: (i, 0, 0)),
            pl.BlockSpec((_BB, _S, _D), lambda i, pos: (i, 0, 0)),
            pl.BlockSpec((_BB, _Q, _D), lambda i, pos: (i, 0, 0)),
            pl.BlockSpec((_BB, _Q, _D), lambda i, pos: (i, 0, 0)),
        ],
        out_specs=[
            pl.BlockSpec((_BB, _S, _D), lambda i, pos: (i, 0, 0)),
            pl.BlockSpec((_BB, _S, _D), lambda i, pos: (i, 0, 0)),
        ],
    )
    ko, vo = pl.pallas_call(
        _body,
        grid_spec=grid_spec,
        out_shape=[
            jax.ShapeDtypeStruct((_BH, _S, _D), k_cache.dtype),
            jax.ShapeDtypeStruct((_BH, _S, _D), v_cache.dtype),
        ],
    )(input_pos, kc, vc, kv, vv)
    return ko.reshape(_B, _H, _S, _D), vo.reshape(_B, _H, _S, _D)
